# baseline (speedup 1.0000x reference)
import functools

import jax
import jax.numpy as jnp
from jax import lax
from jax.experimental import pallas as pl
from jax.experimental.pallas import tpu as pltpu
from jax.experimental.pallas import tpu_sc as plsc

D = 64
SCALE = 8.0

_NC, _NS, _L = 2, 16, 16
_NW = _NC * _NS

IDXW = 128
CHUNK_ROWS = 1024
IDX_ROWS = CHUNK_ROWS // IDXW


@functools.lru_cache(maxsize=None)
def _make_kernel(B, V):
    assert B % (_NW * CHUNK_ROWS) == 0
    b_per_w = B // _NW
    n_chunks = b_per_w // CHUNK_ROWS
    mesh = plsc.VectorSubcoreMesh(
        core_axis_name="c", subcore_axis_name="s",
        num_cores=_NC, num_subcores=_NS,
    )

    @functools.partial(
        pl.kernel,
        out_type=jax.ShapeDtypeStruct((B, D), jnp.float32),
        mesh=mesh,
        scratch_types=[
            pltpu.VMEM((IDX_ROWS, IDXW), jnp.int32),
            pltpu.VMEM((CHUNK_ROWS, D), jnp.float32),
            pltpu.SemaphoreType.DMA,
        ],
        compiler_params=pltpu.CompilerParams(use_tc_tiling_on_sc=False),
    )
    def emb_kernel(idx_hbm, table_hbm, out_hbm, idx_v, rows_v, sem):
        wid = lax.axis_index("s") * _NC + lax.axis_index("c")
        row0 = wid * (b_per_w // IDXW)

        def chunk_body(c, carry):
            g0 = row0 + c * IDX_ROWS
            pltpu.sync_copy(idx_hbm.at[pl.ds(g0, IDX_ROWS)], idx_v)
            copies = [
                pltpu.async_copy(
                    table_hbm.at[idx_v.at[j]],
                    rows_v.at[pl.ds(j * IDXW, IDXW)],
                    sem,
                )
                for j in range(IDX_ROWS)
            ]
            for cp in copies:
                cp.wait()

            def scale_body(r, c2):
                for q in range(D // _L):
                    sl = pl.ds(q * _L, _L)
                    rows_v[r, sl] = rows_v[r, sl] * SCALE
                return c2

            lax.fori_loop(0, CHUNK_ROWS, scale_body, 0)
            pltpu.sync_copy(rows_v, out_hbm.at[pl.ds(g0 * IDXW, CHUNK_ROWS)])
            return carry

        lax.fori_loop(0, n_chunks, chunk_body, 0)

    return emb_kernel


def kernel(tokens, embed_table):
    s0, s1 = tokens.shape
    b = s0 * s1
    idx = tokens.reshape(b // IDXW, IDXW).astype(jnp.int32)
    out = _make_kernel(b, embed_table.shape[0])(idx, embed_table)
    return out.reshape(s0, s1, D)


# speedup vs baseline: 1.0956x; 1.0956x over previous
import functools

import jax
import jax.numpy as jnp
from jax import lax
from jax.experimental import pallas as pl
from jax.experimental.pallas import tpu as pltpu
from jax.experimental.pallas import tpu_sc as plsc

D = 64
SCALE = 8.0

_NC, _NS, _L = 2, 16, 16
_NW = _NC * _NS

IDXW = 128
CHUNK = 256
IDX_R = CHUNK // IDXW
NBUF = 3
ROW_UNROLL = 8


@functools.lru_cache(maxsize=None)
def _make_kernel(B, V):
    assert B % (_NW * CHUNK * NBUF) == 0 or B % (_NW * CHUNK) == 0
    b_per_w = B // _NW
    n_chunks = b_per_w // CHUNK
    assert n_chunks % NBUF == 1
    mesh = plsc.VectorSubcoreMesh(
        core_axis_name="c", subcore_axis_name="s",
        num_cores=_NC, num_subcores=_NS,
    )

    @functools.partial(
        pl.kernel,
        out_type=jax.ShapeDtypeStruct((B, D), jnp.float32),
        mesh=mesh,
        scratch_types=[
            pltpu.VMEM((NBUF, IDX_R, IDXW), jnp.int32),
            pltpu.VMEM((NBUF, CHUNK, D), jnp.float32),
            pltpu.SemaphoreType.DMA((NBUF,)),
            pltpu.SemaphoreType.DMA((NBUF,)),
            pltpu.SemaphoreType.DMA((NBUF,)),
        ],
        compiler_params=pltpu.CompilerParams(use_tc_tiling_on_sc=False),
    )
    def emb_kernel(idx_hbm, table_hbm, out_hbm, idx_v, rows_v, gsem, isem, osem):
        wid = lax.axis_index("s") * _NC + lax.axis_index("c")
        row0 = wid * (b_per_w // IDXW)
        out0 = wid * b_per_w

        def idx_src(cc):
            return idx_hbm.at[pl.ds(row0 + cc * IDX_R, IDX_R)]

        def out_dst(cc):
            return out_hbm.at[pl.ds(out0 + cc * CHUNK, CHUNK)]

        def gather_descs(b):
            return [
                pltpu.make_async_copy(
                    table_hbm.at[idx_v.at[b, j]],
                    rows_v.at[b, pl.ds(j * IDXW, IDXW)],
                    gsem.at[b],
                )
                for j in range(IDX_R)
            ]

        def scale(b):
            def body(r, carry):
                base = r * ROW_UNROLL
                for u in range(ROW_UNROLL):
                    for q in range(D // _L):
                        sl = pl.ds(q * _L, _L)
                        rows_v[b, base + u, sl] = rows_v[b, base + u, sl] * SCALE
                return carry
            lax.fori_loop(0, CHUNK // ROW_UNROLL, body, 0)

        for b in range(NBUF):
            pltpu.async_copy(idx_src(b), idx_v.at[b], isem.at[b])
        pltpu.make_async_copy(idx_src(0), idx_v.at[0], isem.at[0]).wait()
        for d in gather_descs(0):
            d.start()

        def tri_body(i, carry):
            for k in range(NBUF):
                cc = i * NBUF + k
                b, bn = k, (k + 1) % NBUF
                for d2 in gather_descs(b):
                    d2.wait()
                pltpu.make_async_copy(
                    idx_src(cc + 1), idx_v.at[bn], isem.at[bn]
                ).wait()
                @pl.when(cc >= NBUF - 1)
                def _():
                    pltpu.make_async_copy(
                        rows_v.at[bn], out_dst(cc), osem.at[bn]
                    ).wait()
                for d2 in gather_descs(bn):
                    d2.start()
                scale(b)
                pltpu.async_copy(rows_v.at[b], out_dst(cc), osem.at[b])
                @pl.when(cc + NBUF < n_chunks)
                def _():
                    pltpu.async_copy(idx_src(cc + NBUF), idx_v.at[b], isem.at[b])
            return carry

        lax.fori_loop(0, (n_chunks - 1) // NBUF, tri_body, 0)

        last = n_chunks - 1
        bl = last % NBUF
        for d in gather_descs(bl):
            d.wait()
        scale(bl)
        pltpu.async_copy(rows_v.at[bl], out_dst(last), osem.at[bl])
        for b in range(NBUF):
            pltpu.make_async_copy(
                rows_v.at[b], out_dst(last), osem.at[b]
            ).wait()

    return emb_kernel


def kernel(tokens, embed_table):
    s0, s1 = tokens.shape
    b = s0 * s1
    idx = tokens.reshape(b // IDXW, IDXW).astype(jnp.int32)
    out = _make_kernel(b, embed_table.shape[0])(idx, embed_table)
    return out.reshape(s0, s1, D)
